# Initial kernel scaffold; baseline (speedup 1.0000x reference)
import jax
import jax.numpy as jnp
from jax.experimental import pallas as pl


def kernel(query, keys, values, k):
    raise NotImplementedError("write your pallas kernel here")


# speedup vs baseline: 4.4702x; 4.4702x over previous
import functools

import jax
import jax.numpy as jnp
from jax import lax
from jax.experimental import pallas as pl
from jax.experimental.pallas import tpu as pltpu
from jax.experimental.pallas import tpu_sc as plsc

K = 16
BLK = 2048
CHUNK = 128
EPS = 1e-12
NEG_INF = float("-inf")


def _simtopk_body(nblk, cap, q_ref, k_ref, sim_ref, cmax_ref, qn_ref):
    j = pl.program_id(0)

    @pl.when(j == 0)
    def _():
        q = q_ref[...]
        qn = jnp.sqrt(jnp.sum(q * q, axis=1, keepdims=True))
        qn_ref[...] = q / jnp.maximum(qn, EPS)

    kblk = k_ref[...]
    knorm = jnp.sqrt(jnp.sum(kblk * kblk, axis=1, keepdims=True))
    kn = kblk / jnp.maximum(knorm, EPS)
    s = lax.dot_general(
        qn_ref[...].astype(jnp.bfloat16), kn.astype(jnp.bfloat16),
        dimension_numbers=(((1,), (1,)), ((), ())),
        preferred_element_type=jnp.float32,
    )
    col = j * BLK + lax.broadcasted_iota(jnp.int32, s.shape, 1)
    s = jnp.where(col < cap, s, NEG_INF)
    sim_ref[...] = s
    qn_rows = s.shape[0]
    cm = jnp.max(s.reshape(qn_rows, BLK // CHUNK, CHUNK), axis=2)
    cmax_ref[0] = cm


def _sim_and_chunkmax(query, keys_p, cap):
    qn_rows, d = query.shape
    cpad = keys_p.shape[0]
    nblk = cpad // BLK
    return pl.pallas_call(
        functools.partial(_simtopk_body, nblk, cap),
        grid=(nblk,),
        in_specs=[
            pl.BlockSpec((qn_rows, d), lambda j: (0, 0)),
            pl.BlockSpec((BLK, d), lambda j: (j, 0)),
        ],
        out_specs=[
            pl.BlockSpec((qn_rows, BLK), lambda j: (0, j)),
            pl.BlockSpec((1, qn_rows, BLK // CHUNK), lambda j: (j, 0, 0)),
        ],
        out_shape=[
            jax.ShapeDtypeStruct((qn_rows, cpad), jnp.float32),
            jax.ShapeDtypeStruct((nblk, qn_rows, BLK // CHUNK), jnp.float32),
        ],
        scratch_shapes=[pltpu.VMEM((qn_rows, d), jnp.float32)],
        compiler_params=pltpu.CompilerParams(
            dimension_semantics=("arbitrary",),
        ),
    )(query, keys_p)


def _chunk_select_body(cmax_ref, ids_ref):
    x = cmax_ref[...]
    shp = x.shape
    cid = (lax.broadcasted_iota(jnp.int32, shp, 0) * (BLK // CHUNK)
           + lax.broadcasted_iota(jnp.int32, shp, 2))
    big = jnp.int32(2147483647)
    for t in range(K):
        m2 = jnp.max(x, axis=0)
        m = jnp.max(m2, axis=1, keepdims=True)
        cand = jnp.where(x == m[None], cid, big)
        s2 = jnp.min(cand, axis=0)
        sel = jnp.min(s2, axis=1, keepdims=True)
        ids_ref[:, pl.ds(t, 1)] = sel
        x = jnp.where(cid == sel[None], NEG_INF, x)


def _chunk_select(cmax3):
    nblk, qn_rows, w = cmax3.shape
    return pl.pallas_call(
        _chunk_select_body,
        out_shape=jax.ShapeDtypeStruct((qn_rows, K), jnp.int32),
    )(cmax3)


def _final_topk_body(cand_ref, gidx_ref, score_ref, idx_ref):
    x = cand_ref[...]
    g = gidx_ref[...]
    big = jnp.int32(2147483647)
    for t in range(K):
        m = jnp.max(x, axis=1, keepdims=True)
        sel = jnp.min(jnp.where(x == m, g, big), axis=1, keepdims=True)
        score_ref[:, pl.ds(t, 1)] = m
        idx_ref[:, pl.ds(t, 1)] = sel
        x = jnp.where(g == sel, NEG_INF, x)


def _final_topk(cand, gidx):
    qn_rows = cand.shape[0]
    return pl.pallas_call(
        _final_topk_body,
        out_shape=[
            jax.ShapeDtypeStruct((qn_rows, K), jnp.float32),
            jax.ShapeDtypeStruct((qn_rows, K), jnp.int32),
        ],
    )(cand, gidx)


def _sc_gather(table, idx2d):
    nrow_blocks = idx2d.shape[0]
    d = table.shape[1]
    nw = 32
    pwb = nrow_blocks // nw
    mesh = plsc.VectorSubcoreMesh(core_axis_name="c", subcore_axis_name="s")

    @functools.partial(
        pl.kernel,
        mesh=mesh,
        out_type=jax.ShapeDtypeStruct((nrow_blocks * 128, d), jnp.float32),
        scratch_types=[
            pltpu.VMEM((pwb, 128), jnp.int32),
            pltpu.VMEM((128, d), jnp.float32),
            pltpu.SemaphoreType.DMA,
        ],
    )
    def gk(table_hbm, idx_hbm, out_hbm, idx_v, rows_v, sem):
        cidx = lax.axis_index("c")
        sidx = lax.axis_index("s")
        wid = sidx * 2 + cidx
        blk0 = wid * pwb
        pltpu.sync_copy(idx_hbm.at[pl.ds(blk0, pwb)], idx_v)
        for jj in range(pwb):
            pltpu.async_copy(table_hbm.at[idx_v.at[jj]], rows_v, sem).wait()
            pltpu.sync_copy(rows_v, out_hbm.at[pl.ds((blk0 + jj) * 128, 128)])

    return gk(table, idx2d)


def kernel(query, keys, values, k):
    qn_rows, d = query.shape
    cap = keys.shape[0]
    dv = values.shape[1]

    cpad = ((cap + BLK - 1) // BLK) * BLK
    keys_p = jnp.pad(keys, ((0, cpad - cap), (0, 0)))
    nchunks = cpad // CHUNK

    sim, cmax3 = _sim_and_chunkmax(query, keys_p, cap)
    chunk_ids = _chunk_select(cmax3)

    flat_chunk = (jnp.arange(qn_rows, dtype=jnp.int32)[:, None] * nchunks
                  + chunk_ids).reshape(-1)
    cand = _sc_gather(sim.reshape(qn_rows * nchunks, CHUNK),
                      flat_chunk.reshape(-1, 128))
    cand = cand.reshape(qn_rows, K * CHUNK)
    gidx = (chunk_ids[:, :, None] * CHUNK
            + jnp.arange(CHUNK, dtype=jnp.int32)[None, None, :]
            ).reshape(qn_rows, K * CHUNK)

    scores, indices = _final_topk(cand, gidx)

    rows = _sc_gather(values, indices.reshape(-1, 128))
    retrieved = rows.reshape(qn_rows, K, dv)
    return retrieved, scores


# speedup vs baseline: 4.6651x; 1.0436x over previous
import functools

import jax
import jax.numpy as jnp
from jax import lax
from jax.experimental import pallas as pl
from jax.experimental.pallas import tpu as pltpu
from jax.experimental.pallas import tpu_sc as plsc

K = 16
BLK = 2048
CHUNK = 128
NCPB = BLK // CHUNK
EPS = 1e-12
NEG_INF = float("-inf")


def _simtopk_body(nblk, cap, q_ref, k_ref, sim_ref, cmax_ref, bids_ref,
                  qn_ref, bmax_ref):
    j = pl.program_id(0)
    qrows = q_ref.shape[0]

    @pl.when(j == 0)
    def _():
        q = q_ref[...]
        qn = jnp.sqrt(jnp.sum(q * q, axis=1, keepdims=True))
        qn_ref[...] = q / jnp.maximum(qn, EPS)
        bmax_ref[...] = jnp.full(bmax_ref.shape, NEG_INF, jnp.float32)

    kblk = k_ref[...]
    knorm = jnp.sqrt(jnp.sum(kblk * kblk, axis=1, keepdims=True))
    kn = kblk / jnp.maximum(knorm, EPS)
    s = lax.dot_general(
        qn_ref[...].astype(jnp.bfloat16), kn.astype(jnp.bfloat16),
        dimension_numbers=(((1,), (1,)), ((), ())),
        preferred_element_type=jnp.float32,
    )
    col = j * BLK + lax.broadcasted_iota(jnp.int32, s.shape, 1)
    s = jnp.where(col < cap, s, NEG_INF)
    sim_ref[...] = s
    cm = jnp.max(s.reshape(qrows, NCPB, CHUNK), axis=2)
    cmax_ref[0] = jnp.full((qrows, 128), NEG_INF, jnp.float32)
    cmax_ref[0, :, 0:NCPB] = cm
    bm = jnp.max(cm, axis=1, keepdims=True)
    lane = lax.broadcasted_iota(jnp.int32, bmax_ref.shape, 1)
    bmax_ref[...] = jnp.where(lane == j, bm, bmax_ref[...])

    @pl.when(j == nblk - 1)
    def _():
        x = bmax_ref[...]
        big = jnp.int32(2147483647)
        for t in range(K):
            m = jnp.max(x, axis=1, keepdims=True)
            sel = jnp.min(jnp.where(x == m, lane, big), axis=1, keepdims=True)
            bids_ref[:, pl.ds(t, 1)] = sel
            x = jnp.where(lane == sel, NEG_INF, x)


def _sim_and_blocktopk(query, keys_p, cap):
    qrows, d = query.shape
    cpad = keys_p.shape[0]
    nblk = cpad // BLK
    return pl.pallas_call(
        functools.partial(_simtopk_body, nblk, cap),
        grid=(nblk,),
        in_specs=[
            pl.BlockSpec((qrows, d), lambda j: (0, 0)),
            pl.BlockSpec((BLK, d), lambda j: (j, 0)),
        ],
        out_specs=[
            pl.BlockSpec((qrows, BLK), lambda j: (0, j)),
            pl.BlockSpec((1, qrows, 128), lambda j: (j, 0, 0)),
            pl.BlockSpec((qrows, K), lambda j: (0, 0)),
        ],
        out_shape=[
            jax.ShapeDtypeStruct((qrows, cpad), jnp.float32),
            jax.ShapeDtypeStruct((nblk, qrows, 128), jnp.float32),
            jax.ShapeDtypeStruct((qrows, K), jnp.int32),
        ],
        scratch_shapes=[
            pltpu.VMEM((qrows, d), jnp.float32),
            pltpu.VMEM((qrows, 128), jnp.float32),
        ],
        compiler_params=pltpu.CompilerParams(
            dimension_semantics=("arbitrary",),
        ),
    )(query, keys_p)


def _select_body(x_ref, g_ref, val_ref, idx_ref):
    x = x_ref[...]
    g = g_ref[...]
    big = jnp.int32(2147483647)
    for t in range(K):
        m = jnp.max(x, axis=1, keepdims=True)
        sel = jnp.min(jnp.where(x == m, g, big), axis=1, keepdims=True)
        val_ref[:, pl.ds(t, 1)] = m
        idx_ref[:, pl.ds(t, 1)] = sel
        x = jnp.where(g == sel, NEG_INF, x)


def _select_topk(x, g):
    qrows = x.shape[0]
    return pl.pallas_call(
        _select_body,
        out_shape=[
            jax.ShapeDtypeStruct((qrows, K), jnp.float32),
            jax.ShapeDtypeStruct((qrows, K), jnp.int32),
        ],
    )(x, g)


def _sc_gather(table, idx2d):
    nrow_blocks = idx2d.shape[0]
    d = table.shape[1]
    nw = 32
    pwb = nrow_blocks // nw
    mesh = plsc.VectorSubcoreMesh(core_axis_name="c", subcore_axis_name="s")

    @functools.partial(
        pl.kernel,
        mesh=mesh,
        out_type=jax.ShapeDtypeStruct((nrow_blocks * 128, d), jnp.float32),
        scratch_types=[
            pltpu.VMEM((pwb, 128), jnp.int32),
            pltpu.VMEM((128, d), jnp.float32),
            pltpu.SemaphoreType.DMA,
        ],
    )
    def gk(table_hbm, idx_hbm, out_hbm, idx_v, rows_v, sem):
        cidx = lax.axis_index("c")
        sidx = lax.axis_index("s")
        wid = sidx * 2 + cidx
        blk0 = wid * pwb
        pltpu.sync_copy(idx_hbm.at[pl.ds(blk0, pwb)], idx_v)
        for jj in range(pwb):
            pltpu.async_copy(table_hbm.at[idx_v.at[jj]], rows_v, sem).wait()
            pltpu.sync_copy(rows_v, out_hbm.at[pl.ds((blk0 + jj) * 128, 128)])

    return gk(table, idx2d)


def kernel(query, keys, values, k):
    qrows, d = query.shape
    cap = keys.shape[0]
    dv = values.shape[1]

    cpad = ((cap + BLK - 1) // BLK) * BLK
    keys_p = jnp.pad(keys, ((0, cpad - cap), (0, 0)))
    nblk = cpad // BLK
    nchunks = cpad // CHUNK

    sim, cmax3, block_ids = _sim_and_blocktopk(query, keys_p, cap)

    g1_idx = (block_ids * qrows
              + jnp.arange(qrows, dtype=jnp.int32)[:, None]).reshape(-1)
    candmax = _sc_gather(cmax3.reshape(nblk * qrows, 128),
                         g1_idx.reshape(-1, 128))
    candmax = candmax.reshape(qrows, K * 128)
    gcid = (block_ids[:, :, None] * NCPB
            + (jnp.arange(128, dtype=jnp.int32) % NCPB)[None, None, :]
            ).reshape(qrows, K * 128)
    _, chunk_ids = _select_topk(candmax, gcid)

    g2_idx = (jnp.arange(qrows, dtype=jnp.int32)[:, None] * nchunks
              + chunk_ids).reshape(-1)
    cand = _sc_gather(sim.reshape(qrows * nchunks, CHUNK),
                      g2_idx.reshape(-1, 128))
    cand = cand.reshape(qrows, K * CHUNK)
    gidx = (chunk_ids[:, :, None] * CHUNK
            + jnp.arange(CHUNK, dtype=jnp.int32)[None, None, :]
            ).reshape(qrows, K * CHUNK)
    scores, indices = _select_topk(cand, gidx)

    rows = _sc_gather(values, indices.reshape(-1, 128))
    retrieved = rows.reshape(qrows, K, dv)
    return retrieved, scores


# speedup vs baseline: 6.8431x; 1.4669x over previous
import functools

import jax
import jax.numpy as jnp
from jax import lax
from jax.experimental import pallas as pl
from jax.experimental.pallas import tpu as pltpu
from jax.experimental.pallas import tpu_sc as plsc

K = 16
BLK = 2048
CHUNK = 128
NCPB = BLK // CHUNK
EPS = 1e-12
NEG_INF = float("-inf")


def _simtopk_body(nblk, cap, q_ref, k_ref, sim_ref, cmax_ref, bids_ref,
                  qn_ref, bmax_ref):
    j = pl.program_id(0)
    qrows = q_ref.shape[0]

    @pl.when(j == 0)
    def _():
        q = q_ref[...]
        qn = jnp.sqrt(jnp.sum(q * q, axis=1, keepdims=True))
        qn_ref[...] = (q / jnp.maximum(qn, EPS)).astype(jnp.bfloat16)
        bmax_ref[...] = jnp.full(bmax_ref.shape, NEG_INF, jnp.float32)

    kblk = k_ref[...]
    knorm = jnp.sqrt(jnp.sum(kblk * kblk, axis=1, keepdims=True))
    kn = kblk / jnp.maximum(knorm, EPS)
    s = lax.dot_general(
        qn_ref[...], kn.astype(jnp.bfloat16),
        dimension_numbers=(((1,), (1,)), ((), ())),
        preferred_element_type=jnp.float32,
    )
    col = j * BLK + lax.broadcasted_iota(jnp.int32, s.shape, 1)
    s3 = jnp.where(col < cap, s, NEG_INF).reshape(qrows, NCPB, CHUNK)
    sim_ref[...] = s3
    cm = jnp.max(s3, axis=2)
    cmax_ref[0] = jnp.full((qrows, 128), NEG_INF, jnp.float32)
    cmax_ref[0, :, 0:NCPB] = cm
    bm = jnp.max(cm, axis=1, keepdims=True)
    lane = lax.broadcasted_iota(jnp.int32, bmax_ref.shape, 1)
    bmax_ref[...] = jnp.where(lane == j, bm, bmax_ref[...])

    @pl.when(j == nblk - 1)
    def _():
        x = bmax_ref[...]
        big = jnp.int32(2147483647)
        for t in range(K):
            m = jnp.max(x, axis=1, keepdims=True)
            sel = jnp.min(jnp.where(x == m, lane, big), axis=1, keepdims=True)
            bids_ref[:, pl.ds(t, 1)] = sel
            x = jnp.where(lane == sel, NEG_INF, x)


def _sim_and_blocktopk(query, keys_p, cap):
    qrows, d = query.shape
    cpad = keys_p.shape[0]
    nblk = cpad // BLK
    return pl.pallas_call(
        functools.partial(_simtopk_body, nblk, cap),
        grid=(nblk,),
        in_specs=[
            pl.BlockSpec((qrows, d), lambda j: (0, 0)),
            pl.BlockSpec((BLK, d), lambda j: (j, 0)),
        ],
        out_specs=[
            pl.BlockSpec((qrows, NCPB, CHUNK), lambda j: (0, j, 0)),
            pl.BlockSpec((1, qrows, 128), lambda j: (j, 0, 0)),
            pl.BlockSpec((qrows, K), lambda j: (0, 0)),
        ],
        out_shape=[
            jax.ShapeDtypeStruct((qrows, cpad // CHUNK, CHUNK), jnp.float32),
            jax.ShapeDtypeStruct((nblk, qrows, 128), jnp.float32),
            jax.ShapeDtypeStruct((qrows, K), jnp.int32),
        ],
        scratch_shapes=[
            pltpu.VMEM((qrows, d), jnp.bfloat16),
            pltpu.VMEM((qrows, 128), jnp.float32),
        ],
        compiler_params=pltpu.CompilerParams(
            dimension_semantics=("arbitrary",),
        ),
    )(query, keys_p)


def _select_body(x_ref, g_ref, val_ref, idx_ref):
    x = x_ref[...]
    g = g_ref[...]
    big = jnp.int32(2147483647)
    for t in range(K):
        m = jnp.max(x, axis=1, keepdims=True)
        sel = jnp.min(jnp.where(x == m, g, big), axis=1, keepdims=True)
        val_ref[:, pl.ds(t, 1)] = m
        idx_ref[:, pl.ds(t, 1)] = sel
        x = jnp.where(g == sel, NEG_INF, x)


def _select_topk(x, g):
    qrows = x.shape[0]
    return pl.pallas_call(
        _select_body,
        out_shape=[
            jax.ShapeDtypeStruct((qrows, K), jnp.float32),
            jax.ShapeDtypeStruct((qrows, K), jnp.int32),
        ],
    )(x, g)


def _sc_gather(table, idx2d):
    nrow_blocks = idx2d.shape[0]
    d = table.shape[1]
    nw = 32
    pwb = nrow_blocks // nw
    mesh = plsc.VectorSubcoreMesh(core_axis_name="c", subcore_axis_name="s")

    @functools.partial(
        pl.kernel,
        mesh=mesh,
        out_type=jax.ShapeDtypeStruct((nrow_blocks * 128, d), jnp.float32),
        scratch_types=[
            pltpu.VMEM((pwb, 128), jnp.int32),
            pltpu.VMEM((128, d), jnp.float32),
            pltpu.SemaphoreType.DMA,
        ],
    )
    def gk(table_hbm, idx_hbm, out_hbm, idx_v, rows_v, sem):
        cidx = lax.axis_index("c")
        sidx = lax.axis_index("s")
        wid = sidx * 2 + cidx
        blk0 = wid * pwb
        pltpu.sync_copy(idx_hbm.at[pl.ds(blk0, pwb)], idx_v)
        for jj in range(pwb):
            pltpu.async_copy(table_hbm.at[idx_v.at[jj]], rows_v, sem).wait()
            pltpu.sync_copy(rows_v, out_hbm.at[pl.ds((blk0 + jj) * 128, 128)])

    return gk(table, idx2d)


def kernel(query, keys, values, k):
    qrows, d = query.shape
    cap = keys.shape[0]
    dv = values.shape[1]

    cpad = ((cap + BLK - 1) // BLK) * BLK
    keys_p = jnp.pad(keys, ((0, cpad - cap), (0, 0)))
    nblk = cpad // BLK
    nchunks = cpad // CHUNK

    sim3, cmax3, block_ids = _sim_and_blocktopk(query, keys_p, cap)

    g1_idx = (block_ids * qrows
              + jnp.arange(qrows, dtype=jnp.int32)[:, None]).reshape(-1)
    candmax = _sc_gather(cmax3.reshape(nblk * qrows, 128),
                         g1_idx.reshape(-1, 128))
    candmax = candmax.reshape(qrows, K * 128)
    gcid = (block_ids[:, :, None] * NCPB
            + (jnp.arange(128, dtype=jnp.int32) % NCPB)[None, None, :]
            ).reshape(qrows, K * 128)
    _, chunk_ids = _select_topk(candmax, gcid)

    g2_idx = (jnp.arange(qrows, dtype=jnp.int32)[:, None] * nchunks
              + chunk_ids).reshape(-1)
    cand = _sc_gather(sim3.reshape(qrows * nchunks, CHUNK),
                      g2_idx.reshape(-1, 128))
    cand = cand.reshape(qrows, K * CHUNK)
    gidx = (chunk_ids[:, :, None] * CHUNK
            + jnp.arange(CHUNK, dtype=jnp.int32)[None, None, :]
            ).reshape(qrows, K * CHUNK)
    scores, indices = _select_topk(cand, gidx)

    rows = _sc_gather(values, indices.reshape(-1, 128))
    retrieved = rows.reshape(qrows, K, dv)
    return retrieved, scores


# speedup vs baseline: 8.6862x; 1.2693x over previous
import functools

import jax
import jax.numpy as jnp
from jax import lax
from jax.experimental import pallas as pl
from jax.experimental.pallas import tpu as pltpu
from jax.experimental.pallas import tpu_sc as plsc

K = 16
BLK = 2048
CHUNK = 128
NCPB = BLK // CHUNK
EPS = 1e-12
NEG_INF = float("-inf")


def _simtopk_body(nblk, cap, q_ref, k_ref, sim_ref, cmax_ref, bids_ref,
                  qn_ref, bmax_ref):
    j = pl.program_id(0)
    qrows = q_ref.shape[0]

    @pl.when(j == 0)
    def _():
        q = q_ref[...]
        qn = jnp.sqrt(jnp.sum(q * q, axis=1, keepdims=True))
        qn_ref[...] = (q / jnp.maximum(qn, EPS)).astype(jnp.bfloat16)
        bmax_ref[...] = jnp.full(bmax_ref.shape, NEG_INF, jnp.float32)

    kblk = k_ref[...]
    knorm = jnp.sqrt(jnp.sum(kblk * kblk, axis=1, keepdims=True))
    kn = kblk / jnp.maximum(knorm, EPS)
    s = lax.dot_general(
        qn_ref[...], kn.astype(jnp.bfloat16),
        dimension_numbers=(((1,), (1,)), ((), ())),
        preferred_element_type=jnp.float32,
    )
    col = j * BLK + lax.broadcasted_iota(jnp.int32, s.shape, 1)
    s3 = jnp.where(col < cap, s, NEG_INF).reshape(qrows, NCPB, CHUNK)
    sim_ref[...] = s3
    cm = jnp.max(s3, axis=2)
    cmax_ref[0] = jnp.full((qrows, 128), NEG_INF, jnp.float32)
    cmax_ref[0, :, 0:NCPB] = cm
    bm = jnp.max(cm, axis=1, keepdims=True)
    lane = lax.broadcasted_iota(jnp.int32, bmax_ref.shape, 1)
    bmax_ref[...] = jnp.where(lane == j, bm, bmax_ref[...])

    @pl.when(j == nblk - 1)
    def _():
        x = bmax_ref[...]
        big = jnp.int32(2147483647)
        for t in range(K):
            m = jnp.max(x, axis=1, keepdims=True)
            sel = jnp.min(jnp.where(x == m, lane, big), axis=1, keepdims=True)
            bids_ref[:, pl.ds(t, 1)] = sel
            x = jnp.where(lane == sel, NEG_INF, x)


def _sim_and_blocktopk(query, keys, cap):
    qrows, d = query.shape
    cpad = ((cap + BLK - 1) // BLK) * BLK
    nblk = cpad // BLK
    return pl.pallas_call(
        functools.partial(_simtopk_body, nblk, cap),
        grid=(nblk,),
        in_specs=[
            pl.BlockSpec((qrows, d), lambda j: (0, 0)),
            pl.BlockSpec((BLK, d), lambda j: (j, 0)),
        ],
        out_specs=[
            pl.BlockSpec((qrows, NCPB, CHUNK), lambda j: (0, j, 0)),
            pl.BlockSpec((1, qrows, 128), lambda j: (j, 0, 0)),
            pl.BlockSpec((qrows, K), lambda j: (0, 0)),
        ],
        out_shape=[
            jax.ShapeDtypeStruct((qrows, cpad // CHUNK, CHUNK), jnp.float32),
            jax.ShapeDtypeStruct((nblk, qrows, 128), jnp.float32),
            jax.ShapeDtypeStruct((qrows, K), jnp.int32),
        ],
        scratch_shapes=[
            pltpu.VMEM((qrows, d), jnp.bfloat16),
            pltpu.VMEM((qrows, 128), jnp.float32),
        ],
        compiler_params=pltpu.CompilerParams(
            dimension_semantics=("arbitrary",),
        ),
    )(query, keys)


def _select_body(x_ref, g_ref, val_ref, idx_ref):
    x = x_ref[...]
    g = g_ref[...]
    big = jnp.int32(2147483647)
    for t in range(K):
        m = jnp.max(x, axis=1, keepdims=True)
        sel = jnp.min(jnp.where(x == m, g, big), axis=1, keepdims=True)
        val_ref[:, pl.ds(t, 1)] = m
        idx_ref[:, pl.ds(t, 1)] = sel
        x = jnp.where(g == sel, NEG_INF, x)


def _select_topk(x, g):
    qrows = x.shape[0]
    return pl.pallas_call(
        _select_body,
        out_shape=[
            jax.ShapeDtypeStruct((qrows, K), jnp.float32),
            jax.ShapeDtypeStruct((qrows, K), jnp.int32),
        ],
    )(x, g)


def _sc_gather(table, idx2d):
    nrow_blocks = idx2d.shape[0]
    d = table.shape[1]
    nw = 32
    pwb = nrow_blocks // nw
    mesh = plsc.VectorSubcoreMesh(core_axis_name="c", subcore_axis_name="s")

    @functools.partial(
        pl.kernel,
        mesh=mesh,
        out_type=jax.ShapeDtypeStruct((nrow_blocks * 128, d), jnp.float32),
        scratch_types=[
            pltpu.VMEM((pwb, 128), jnp.int32),
            pltpu.VMEM((128, d), jnp.float32),
            pltpu.SemaphoreType.DMA,
        ],
    )
    def gk(table_hbm, idx_hbm, out_hbm, idx_v, rows_v, sem):
        cidx = lax.axis_index("c")
        sidx = lax.axis_index("s")
        wid = sidx * 2 + cidx
        blk0 = wid * pwb
        pltpu.sync_copy(idx_hbm.at[pl.ds(blk0, pwb)], idx_v)
        for jj in range(pwb):
            pltpu.async_copy(table_hbm.at[idx_v.at[jj]], rows_v, sem).wait()
            pltpu.sync_copy(rows_v, out_hbm.at[pl.ds((blk0 + jj) * 128, 128)])

    return gk(table, idx2d)


def kernel(query, keys, values, k):
    qrows, d = query.shape
    cap = keys.shape[0]
    dv = values.shape[1]

    cpad = ((cap + BLK - 1) // BLK) * BLK
    nblk = cpad // BLK
    nchunks = cpad // CHUNK

    sim3, cmax3, block_ids = _sim_and_blocktopk(query, keys, cap)

    g1_idx = (block_ids * qrows
              + jnp.arange(qrows, dtype=jnp.int32)[:, None]).reshape(-1)
    candmax = _sc_gather(cmax3.reshape(nblk * qrows, 128),
                         g1_idx.reshape(-1, 128))
    candmax = candmax.reshape(qrows, K * 128)
    gcid = (block_ids[:, :, None] * NCPB
            + (jnp.arange(128, dtype=jnp.int32) % NCPB)[None, None, :]
            ).reshape(qrows, K * 128)
    _, chunk_ids = _select_topk(candmax, gcid)

    g2_idx = (jnp.arange(qrows, dtype=jnp.int32)[:, None] * nchunks
              + chunk_ids).reshape(-1)
    cand = _sc_gather(sim3.reshape(qrows * nchunks, CHUNK),
                      g2_idx.reshape(-1, 128))
    cand = cand.reshape(qrows, K * CHUNK)
    gidx = (chunk_ids[:, :, None] * CHUNK
            + jnp.arange(CHUNK, dtype=jnp.int32)[None, None, :]
            ).reshape(qrows, K * CHUNK)
    scores, indices = _select_topk(cand, gidx)

    rows = _sc_gather(values, indices.reshape(-1, 128))
    retrieved = rows.reshape(qrows, K, dv)
    return retrieved, scores
